# baseline (speedup 1.0000x reference)
import functools

import jax
import jax.numpy as jnp
from jax import lax
from jax.experimental import pallas as pl
from jax.experimental.pallas import tpu as pltpu
from jax.experimental.pallas import tpu_sc as plsc

N = 10000
E = 320000
D = 128
CHUNK = 128
NC, NS = 2, 16
NJ = 79
NCHUNKP = NC * NS * NJ
EPAD = NCHUNKP * CHUNK - E
NPAD = 10240
DPT = NPAD // NS

BN = 1000
GRID = N // BN

_MESH = plsc.VectorSubcoreMesh(
    core_axis_name="c", subcore_axis_name="s", num_cores=NC, num_subcores=NS)


def _zero_1d(ref, n):
    for j in range(n // 16):
        ref[pl.ds(16 * j, 16)] = jnp.zeros((16,), jnp.float32)


def _deg_body(ei_hbm, out_hbm, idxb, ones_v, zbuf, sem_s, acc):
    c = lax.axis_index("c")
    s = lax.axis_index("s")
    w = c * NS + s
    for j in range(8):
        ones_v[pl.ds(16 * j, 16)] = jnp.ones((16,), jnp.float32)
    _zero_1d(zbuf, DPT)
    pltpu.sync_copy(zbuf, acc.at[pl.ds(s * DPT, DPT)])
    pltpu.sync_copy(ei_hbm.at[pl.ds(w * NJ, NJ)], idxb)
    plsc.subcore_barrier()

    def body(j, carry):
        cp = pltpu.async_copy(ones_v, acc.at[idxb.at[j, 1]], sem_s, add=True)
        return carry

    lax.fori_loop(0, NJ, body, 0)

    def drain(j, carry):
        pltpu.make_async_copy(ones_v, acc.at[idxb.at[j, 1]], sem_s).wait()
        return carry

    lax.fori_loop(0, NJ, drain, 0)
    plsc.subcore_barrier()
    pltpu.sync_copy(acc.at[pl.ds(s * DPT, DPT)],
                    out_hbm.at[c, pl.ds(s * DPT, DPT)])


_deg_call = pl.kernel(
    _deg_body,
    out_type=jax.ShapeDtypeStruct((NC, NPAD), jnp.float32),
    mesh=_MESH,
    scratch_types=[
        pltpu.VMEM((NJ, 2, CHUNK), jnp.int32),
        pltpu.VMEM((CHUNK,), jnp.float32),
        pltpu.VMEM((DPT,), jnp.float32),
        pltpu.SemaphoreType.DMA,
        pltpu.VMEM_SHARED((NPAD,), jnp.float32),
    ],
)


NJ0 = 49
NJ1 = 2 * NJ - NJ0


def _make_agg(ntab, with_flags=False):
    def body(*refs):
        gs = refs[:ntab]
        ei_hbm = refs[ntab]
        out_hbm = refs[ntab + 1]
        idxr, rows, zbuf, sem_i, sem_g, sem_s, acc = refs[ntab + 2:]
        c = lax.axis_index("c")
        s = lax.axis_index("s")
        w = c * NS + s

        def zrow(i, carry):
            for j in range(8):
                zbuf[i, pl.ds(16 * j, 16)] = jnp.zeros((16,), jnp.float32)
            return carry

        lax.fori_loop(0, 64, zrow, 0)

        for t in range(ntab):
            for k in range(DPT // 64):
                pltpu.sync_copy(zbuf, acc.at[pl.ds(s * DPT + k * 64, 64)])
            plsc.subcore_barrier()
            g = gs[t]

            def _sweep(g=g):
                njc = jnp.where(c == 0, NJ0, NJ1)
                cb = jnp.where(c == 0, s, NS * NJ0 + s)
                pltpu.sync_copy(ei_hbm.at[cb], idxr.at[0])
                pltpu.async_copy(g.at[idxr.at[0, 0]], rows.at[0], sem_g)
                pltpu.async_copy(ei_hbm.at[cb + NS], idxr.at[1], sem_i)

                def body_j(j, carry, g=g):
                    b = lax.rem(j, 2)
                    nb = 1 - b
                    sl1 = lax.rem(j + 1, 3)
                    sl2 = lax.rem(j + 2, 3)

                    @pl.when(j + 2 < njc)
                    def _():
                        pltpu.async_copy(
                            ei_hbm.at[cb + NS * (j + 2)], idxr.at[sl2], sem_i)

                    pltpu.make_async_copy(
                        ei_hbm.at[cb + NS * (j + 1)], idxr.at[sl1], sem_i).wait()
                    pltpu.async_copy(g.at[idxr.at[sl1, 0]], rows.at[nb], sem_g)
                    pltpu.make_async_copy(
                        g.at[idxr.at[sl1, 0]], rows.at[b], sem_g).wait()
                    pltpu.sync_copy(
                        rows.at[b], acc.at[idxr.at[lax.rem(j, 3), 1]], add=True)
                    return carry

                lax.fori_loop(0, njc - 1, body_j, 0)
                last = njc - 1
                lb = lax.rem(last, 2)
                slL = lax.rem(last, 3)
                pltpu.make_async_copy(
                    g.at[idxr.at[slL, 0]], rows.at[lb], sem_g).wait()
                pltpu.sync_copy(
                    rows.at[lb], acc.at[idxr.at[slL, 1]], add=True)

            _sweep()
            plsc.subcore_barrier()
            pltpu.sync_copy(acc.at[pl.ds(s * DPT, DPT)],
                            out_hbm.at[t, c, pl.ds(s * DPT, DPT)])
            plsc.subcore_barrier()

    return pl.kernel(
        body,
        out_type=jax.ShapeDtypeStruct((ntab, NC, NPAD, D), jnp.float32),
        mesh=_MESH,
        scratch_types=[
            pltpu.VMEM((3, 2, CHUNK), jnp.int32),
            pltpu.VMEM((2, CHUNK, D), jnp.float32),
            pltpu.VMEM((64, D), jnp.float32),
            pltpu.SemaphoreType.DMA,
            pltpu.SemaphoreType.DMA,
            pltpu.SemaphoreType.DMA,
            pltpu.VMEM_SHARED((NPAD, D), jnp.float32),
        ],
    )


_agg1 = _make_agg(1)
_agg4 = _make_agg(4)

_DOT = dict(preferred_element_type=jnp.float32, precision=lax.Precision.HIGHEST)


def _tc1_body(deg_ref, x_ref, w_ref, b_ref, inv_ref, g1_ref):
    dsum = deg_ref[0] + deg_ref[1] + 1.0
    iv = lax.rsqrt(dsum)
    inv_ref[...] = iv
    hl = jnp.dot(x_ref[...], w_ref[...], **_DOT) + b_ref[...]
    g1_ref[...] = hl * iv


_tc1 = pl.pallas_call(
    _tc1_body,
    grid=(GRID,),
    in_specs=[
        pl.BlockSpec((NC, BN, 1), lambda i: (0, i, 0)),
        pl.BlockSpec((BN, D), lambda i: (i, 0)),
        pl.BlockSpec((D, D), lambda i: (0, 0)),
        pl.BlockSpec((1, D), lambda i: (0, 0)),
    ],
    out_specs=[
        pl.BlockSpec((BN, 1), lambda i: (i, 0)),
        pl.BlockSpec((BN, D), lambda i: (i, 0)),
    ],
    out_shape=[
        jax.ShapeDtypeStruct((N, 1), jnp.float32),
        jax.ShapeDtypeStruct((N, D), jnp.float32),
    ],
)


def _tc2_body(p_ref, g1_ref, inv_ref, w_ref, b_ref, g2_ref, fl_ref):
    i = pl.program_id(0)

    @pl.when(i == 0)
    def _():
        fl_ref[...] = jnp.zeros_like(fl_ref)

    iv = inv_ref[...]
    cur = iv * (p_ref[0] + p_ref[1] + g1_ref[...])
    v = jnp.zeros_like(cur)
    for t in range(4):
        v = v + (cur - v) * 0.5
        sp = (v >= 1.0).astype(jnp.float32)
        v = v - sp
        g2t = (jnp.dot(sp, w_ref[...], **_DOT) + b_ref[...]) * iv
        g2_ref[t] = g2t
        fl_ref[t] = jnp.maximum(fl_ref[t], jnp.max(jnp.abs(g2t)))


_tc2 = pl.pallas_call(
    _tc2_body,
    grid=(GRID,),
    in_specs=[
        pl.BlockSpec((NC, BN, D), lambda i: (0, i, 0)),
        pl.BlockSpec((BN, D), lambda i: (i, 0)),
        pl.BlockSpec((BN, 1), lambda i: (i, 0)),
        pl.BlockSpec((D, D), lambda i: (0, 0)),
        pl.BlockSpec((1, D), lambda i: (0, 0)),
    ],
    out_specs=[
        pl.BlockSpec((4, BN, D), lambda i: (0, i, 0)),
        pl.BlockSpec((4, 8, CHUNK), lambda i: (0, 0, 0)),
    ],
    out_shape=[
        jax.ShapeDtypeStruct((4, N, D), jnp.float32),
        jax.ShapeDtypeStruct((4, 8, CHUNK), jnp.float32),
    ],
)


def _tc3_body(pp_ref, g2_ref, inv_ref, w_ref, b_ref, out_ref):
    iv = inv_ref[...]
    v = jnp.zeros((BN, D), jnp.float32)
    acc = jnp.zeros((BN, D), jnp.float32)
    for t in range(4):
        cur = iv * (pp_ref[t, 0] + pp_ref[t, 1] + g2_ref[t])
        v = v + (cur - v) * 0.5
        sp = (v >= 1.0).astype(jnp.float32)
        v = v - sp
        acc = acc + sp
    out_ref[...] = jnp.dot(acc * 0.25, w_ref[...], **_DOT) + b_ref[...]


_tc3 = pl.pallas_call(
    _tc3_body,
    grid=(GRID,),
    in_specs=[
        pl.BlockSpec((4, NC, BN, D), lambda i: (0, 0, i, 0)),
        pl.BlockSpec((4, BN, D), lambda i: (0, i, 0)),
        pl.BlockSpec((BN, 1), lambda i: (i, 0)),
        pl.BlockSpec((D, D), lambda i: (0, 0)),
        pl.BlockSpec((1, D), lambda i: (0, 0)),
    ],
    out_specs=pl.BlockSpec((BN, D), lambda i: (i, 0)),
    out_shape=jax.ShapeDtypeStruct((N, D), jnp.float32),
)


def kernel(x, edge_index, W0, b0, W1, b1, W_out, b_out):
    srcp = jnp.concatenate(
        [edge_index[0], jnp.zeros((EPAD,), jnp.int32)]).reshape(NCHUNKP, CHUNK)
    pad_dst = N + jax.lax.rem(jnp.arange(EPAD, dtype=jnp.int32), jnp.int32(128))
    dstp = jnp.concatenate(
        [edge_index[1], pad_dst]).reshape(NCHUNKP, CHUNK)
    ei = jnp.stack([srcp, dstp], axis=1)
    degp = _deg_call(ei).reshape(NC, NPAD, 1)
    inv, g1 = _tc1(degp, x, W0, b0.reshape(1, D))
    p1 = _agg1(g1, ei).reshape(NC, NPAD, D)
    g2, fl = _tc2(p1, g1, inv, W1, b1.reshape(1, D))
    pp = lax.cond(
        fl[:, 0, 0].max() > 0.0,
        lambda: _agg4(g2[0], g2[1], g2[2], g2[3], ei),
        lambda: jnp.zeros((4, NC, NPAD, D), jnp.float32),
    )
    return _tc3(pp, g2, inv, W_out, b_out.reshape(1, D))


# speedup vs baseline: 1.0923x; 1.0923x over previous
import functools

import jax
import jax.numpy as jnp
from jax import lax
from jax.experimental import pallas as pl
from jax.experimental.pallas import tpu as pltpu
from jax.experimental.pallas import tpu_sc as plsc

N = 10000
E = 320000
D = 128
CHUNK = 128
NC, NS = 2, 16
NJ = 79
NCHUNKP = NC * NS * NJ
EPAD = NCHUNKP * CHUNK - E
NPAD = 10240
DPT = NPAD // NS

BN = 1000
GRID = N // BN

_MESH = plsc.VectorSubcoreMesh(
    core_axis_name="c", subcore_axis_name="s", num_cores=NC, num_subcores=NS)


def _zero_1d(ref, n):
    for j in range(n // 16):
        ref[pl.ds(16 * j, 16)] = jnp.zeros((16,), jnp.float32)


def _deg_body(ei_hbm, out_hbm, idxb, ones_v, zbuf, sem_s, acc):
    c = lax.axis_index("c")
    s = lax.axis_index("s")
    w = c * NS + s
    for j in range(8):
        ones_v[pl.ds(16 * j, 16)] = jnp.ones((16,), jnp.float32)
    _zero_1d(zbuf, DPT)
    pltpu.sync_copy(zbuf, acc.at[pl.ds(s * DPT, DPT)])
    pltpu.sync_copy(ei_hbm.at[pl.ds(w * NJ, NJ)], idxb)
    plsc.subcore_barrier()

    def body(j, carry):
        cp = pltpu.async_copy(ones_v, acc.at[idxb.at[j, 1]], sem_s, add=True)
        return carry

    lax.fori_loop(0, NJ, body, 0)

    def drain(j, carry):
        pltpu.make_async_copy(ones_v, acc.at[idxb.at[j, 1]], sem_s).wait()
        return carry

    lax.fori_loop(0, NJ, drain, 0)
    plsc.subcore_barrier()
    pltpu.sync_copy(acc.at[pl.ds(s * DPT, DPT)],
                    out_hbm.at[c, pl.ds(s * DPT, DPT)])


_deg_call = pl.kernel(
    _deg_body,
    out_type=jax.ShapeDtypeStruct((NC, NPAD), jnp.float32),
    mesh=_MESH,
    scratch_types=[
        pltpu.VMEM((NJ, 2, CHUNK), jnp.int32),
        pltpu.VMEM((CHUNK,), jnp.float32),
        pltpu.VMEM((DPT,), jnp.float32),
        pltpu.SemaphoreType.DMA,
        pltpu.VMEM_SHARED((NPAD,), jnp.float32),
    ],
)


NJ0 = 109
NJ1 = 2 * NJ - NJ0


def _make_agg(ntab, with_flags=False):
    def body(*refs):
        gs = refs[:ntab]
        ei_hbm = refs[ntab]
        out_hbm = refs[ntab + 1]
        idxr, rows, zbuf, sem_i, sem_g, sem_s, acc = refs[ntab + 2:]
        c = lax.axis_index("c")
        s = lax.axis_index("s")
        w = c * NS + s

        def zrow(i, carry):
            for j in range(8):
                zbuf[i, pl.ds(16 * j, 16)] = jnp.zeros((16,), jnp.float32)
            return carry

        lax.fori_loop(0, 64, zrow, 0)

        for t in range(ntab):
            for k in range(DPT // 64):
                pltpu.sync_copy(zbuf, acc.at[pl.ds(s * DPT + k * 64, 64)])
            plsc.subcore_barrier()
            g = gs[t]

            def _sweep(g=g):
                njc = jnp.where(c == 0, NJ0, NJ1)
                cb = jnp.where(c == 0, s, NS * NJ0 + s)
                pltpu.sync_copy(ei_hbm.at[cb], idxr.at[0])
                pltpu.async_copy(g.at[idxr.at[0, 0]], rows.at[0], sem_g)
                pltpu.async_copy(ei_hbm.at[cb + NS], idxr.at[1], sem_i)

                def body_j(j, carry, g=g):
                    b = lax.rem(j, 2)
                    nb = 1 - b
                    sl1 = lax.rem(j + 1, 3)
                    sl2 = lax.rem(j + 2, 3)

                    @pl.when(j + 2 < njc)
                    def _():
                        pltpu.async_copy(
                            ei_hbm.at[cb + NS * (j + 2)], idxr.at[sl2], sem_i)

                    pltpu.make_async_copy(
                        ei_hbm.at[cb + NS * (j + 1)], idxr.at[sl1], sem_i).wait()
                    pltpu.async_copy(g.at[idxr.at[sl1, 0]], rows.at[nb], sem_g)
                    pltpu.make_async_copy(
                        g.at[idxr.at[sl1, 0]], rows.at[b], sem_g).wait()
                    pltpu.sync_copy(
                        rows.at[b], acc.at[idxr.at[lax.rem(j, 3), 1]], add=True)
                    return carry

                lax.fori_loop(0, njc - 1, body_j, 0)
                last = njc - 1
                lb = lax.rem(last, 2)
                slL = lax.rem(last, 3)
                pltpu.make_async_copy(
                    g.at[idxr.at[slL, 0]], rows.at[lb], sem_g).wait()
                pltpu.sync_copy(
                    rows.at[lb], acc.at[idxr.at[slL, 1]], add=True)

            _sweep()
            plsc.subcore_barrier()
            pltpu.sync_copy(acc.at[pl.ds(s * DPT, DPT)],
                            out_hbm.at[t, c, pl.ds(s * DPT, DPT)])
            plsc.subcore_barrier()

    return pl.kernel(
        body,
        out_type=jax.ShapeDtypeStruct((ntab, NC, NPAD, D), jnp.float32),
        mesh=_MESH,
        scratch_types=[
            pltpu.VMEM((3, 2, CHUNK), jnp.int32),
            pltpu.VMEM((2, CHUNK, D), jnp.float32),
            pltpu.VMEM((64, D), jnp.float32),
            pltpu.SemaphoreType.DMA,
            pltpu.SemaphoreType.DMA,
            pltpu.SemaphoreType.DMA,
            pltpu.VMEM_SHARED((NPAD, D), jnp.float32),
        ],
    )


_agg1 = _make_agg(1)
_agg4 = _make_agg(4)

_DOT = dict(preferred_element_type=jnp.float32, precision=lax.Precision.HIGHEST)


def _tc1_body(deg_ref, x_ref, w_ref, b_ref, inv_ref, g1_ref):
    dsum = deg_ref[0] + deg_ref[1] + 1.0
    iv = lax.rsqrt(dsum)
    inv_ref[...] = iv
    hl = jnp.dot(x_ref[...], w_ref[...], **_DOT) + b_ref[...]
    g1_ref[...] = hl * iv


_tc1 = pl.pallas_call(
    _tc1_body,
    grid=(GRID,),
    in_specs=[
        pl.BlockSpec((NC, BN, 1), lambda i: (0, i, 0)),
        pl.BlockSpec((BN, D), lambda i: (i, 0)),
        pl.BlockSpec((D, D), lambda i: (0, 0)),
        pl.BlockSpec((1, D), lambda i: (0, 0)),
    ],
    out_specs=[
        pl.BlockSpec((BN, 1), lambda i: (i, 0)),
        pl.BlockSpec((BN, D), lambda i: (i, 0)),
    ],
    out_shape=[
        jax.ShapeDtypeStruct((N, 1), jnp.float32),
        jax.ShapeDtypeStruct((N, D), jnp.float32),
    ],
)


def _tc2_body(p_ref, g1_ref, inv_ref, w_ref, b_ref, g2_ref, fl_ref):
    i = pl.program_id(0)

    @pl.when(i == 0)
    def _():
        fl_ref[...] = jnp.zeros_like(fl_ref)

    iv = inv_ref[...]
    cur = iv * (p_ref[0] + p_ref[1] + g1_ref[...])
    v = jnp.zeros_like(cur)
    for t in range(4):
        v = v + (cur - v) * 0.5
        sp = (v >= 1.0).astype(jnp.float32)
        v = v - sp
        g2t = (jnp.dot(sp, w_ref[...], **_DOT) + b_ref[...]) * iv
        g2_ref[t] = g2t
        fl_ref[t] = jnp.maximum(fl_ref[t], jnp.max(jnp.abs(g2t)))


_tc2 = pl.pallas_call(
    _tc2_body,
    grid=(GRID,),
    in_specs=[
        pl.BlockSpec((NC, BN, D), lambda i: (0, i, 0)),
        pl.BlockSpec((BN, D), lambda i: (i, 0)),
        pl.BlockSpec((BN, 1), lambda i: (i, 0)),
        pl.BlockSpec((D, D), lambda i: (0, 0)),
        pl.BlockSpec((1, D), lambda i: (0, 0)),
    ],
    out_specs=[
        pl.BlockSpec((4, BN, D), lambda i: (0, i, 0)),
        pl.BlockSpec((4, 8, CHUNK), lambda i: (0, 0, 0)),
    ],
    out_shape=[
        jax.ShapeDtypeStruct((4, N, D), jnp.float32),
        jax.ShapeDtypeStruct((4, 8, CHUNK), jnp.float32),
    ],
)


def _tc3_body(pp_ref, g2_ref, inv_ref, w_ref, b_ref, out_ref):
    iv = inv_ref[...]
    v = jnp.zeros((BN, D), jnp.float32)
    acc = jnp.zeros((BN, D), jnp.float32)
    for t in range(4):
        cur = iv * (pp_ref[t, 0] + pp_ref[t, 1] + g2_ref[t])
        v = v + (cur - v) * 0.5
        sp = (v >= 1.0).astype(jnp.float32)
        v = v - sp
        acc = acc + sp
    out_ref[...] = jnp.dot(acc * 0.25, w_ref[...], **_DOT) + b_ref[...]


_tc3 = pl.pallas_call(
    _tc3_body,
    grid=(GRID,),
    in_specs=[
        pl.BlockSpec((4, NC, BN, D), lambda i: (0, 0, i, 0)),
        pl.BlockSpec((4, BN, D), lambda i: (0, i, 0)),
        pl.BlockSpec((BN, 1), lambda i: (i, 0)),
        pl.BlockSpec((D, D), lambda i: (0, 0)),
        pl.BlockSpec((1, D), lambda i: (0, 0)),
    ],
    out_specs=pl.BlockSpec((BN, D), lambda i: (i, 0)),
    out_shape=jax.ShapeDtypeStruct((N, D), jnp.float32),
)


def kernel(x, edge_index, W0, b0, W1, b1, W_out, b_out):
    srcp = jnp.concatenate(
        [edge_index[0], jnp.zeros((EPAD,), jnp.int32)]).reshape(NCHUNKP, CHUNK)
    pad_dst = N + jax.lax.rem(jnp.arange(EPAD, dtype=jnp.int32), jnp.int32(128))
    dstp = jnp.concatenate(
        [edge_index[1], pad_dst]).reshape(NCHUNKP, CHUNK)
    ei = jnp.stack([srcp, dstp], axis=1)
    degp = _deg_call(ei).reshape(NC, NPAD, 1)
    inv, g1 = _tc1(degp, x, W0, b0.reshape(1, D))
    p1 = _agg1(g1, ei).reshape(NC, NPAD, D)
    g2, fl = _tc2(p1, g1, inv, W1, b1.reshape(1, D))
    pp = lax.cond(
        fl[:, 0, 0].max() > 0.0,
        lambda: _agg4(g2[0], g2[1], g2[2], g2[3], ei),
        lambda: jnp.zeros((4, NC, NPAD, D), jnp.float32),
    )
    return _tc3(pp, g2, inv, W_out, b_out.reshape(1, D))


# speedup vs baseline: 1.1528x; 1.0554x over previous
import functools

import jax
import jax.numpy as jnp
from jax import lax
from jax.experimental import pallas as pl
from jax.experimental.pallas import tpu as pltpu
from jax.experimental.pallas import tpu_sc as plsc

N = 10000
E = 320000
D = 128
CHUNK = 128
NC, NS = 2, 16
NJ = 79
NCHUNKP = NC * NS * NJ
EPAD = NCHUNKP * CHUNK - E
NPAD = 10240
DPT = NPAD // NS

BN = 1000
GRID = N // BN

_MESH = plsc.VectorSubcoreMesh(
    core_axis_name="c", subcore_axis_name="s", num_cores=NC, num_subcores=NS)


def _zero_1d(ref, n):
    for j in range(n // 16):
        ref[pl.ds(16 * j, 16)] = jnp.zeros((16,), jnp.float32)


def _deg_body(ei_hbm, out_hbm, idxb, ones_v, zbuf, sem_s, acc):
    c = lax.axis_index("c")
    s = lax.axis_index("s")
    w = c * NS + s
    for j in range(8):
        ones_v[pl.ds(16 * j, 16)] = jnp.ones((16,), jnp.float32)
    _zero_1d(zbuf, DPT)
    pltpu.sync_copy(zbuf, acc.at[pl.ds(s * DPT, DPT)])
    pltpu.sync_copy(ei_hbm.at[pl.ds(w * NJ, NJ)], idxb)
    plsc.subcore_barrier()

    def body(j, carry):
        cp = pltpu.async_copy(ones_v, acc.at[idxb.at[j, 1]], sem_s, add=True)
        return carry

    lax.fori_loop(0, NJ, body, 0)

    def drain(j, carry):
        pltpu.make_async_copy(ones_v, acc.at[idxb.at[j, 1]], sem_s).wait()
        return carry

    lax.fori_loop(0, NJ, drain, 0)
    plsc.subcore_barrier()
    pltpu.sync_copy(acc.at[pl.ds(s * DPT, DPT)],
                    out_hbm.at[c, pl.ds(s * DPT, DPT)])


_deg_call = pl.kernel(
    _deg_body,
    out_type=jax.ShapeDtypeStruct((NC, NPAD), jnp.float32),
    mesh=_MESH,
    scratch_types=[
        pltpu.VMEM((NJ, 2, CHUNK), jnp.int32),
        pltpu.VMEM((CHUNK,), jnp.float32),
        pltpu.VMEM((DPT,), jnp.float32),
        pltpu.SemaphoreType.DMA,
        pltpu.VMEM_SHARED((NPAD,), jnp.float32),
    ],
)


NJ0 = 109
NJ1 = 2 * NJ - NJ0


def _make_agg(ntab, with_flags=False):
    def body(*refs):
        gs = refs[:ntab]
        ei_hbm = refs[ntab]
        out_hbm = refs[ntab + 1]
        idxr, rows, zbuf, sem_i, sem_g, sem_s, acc = refs[ntab + 2:]
        c = lax.axis_index("c")
        s = lax.axis_index("s")
        w = c * NS + s

        def zrow(i, carry):
            for j in range(8):
                zbuf[i, pl.ds(16 * j, 16)] = jnp.zeros((16,), jnp.float32)
            return carry

        lax.fori_loop(0, 64, zrow, 0)

        for t in range(ntab):
            for k in range(DPT // 64):
                pltpu.sync_copy(zbuf, acc.at[pl.ds(s * DPT + k * 64, 64)])
            plsc.subcore_barrier()
            g = gs[t]

            def _sweep(g=g):
                njc = jnp.where(c == 0, NJ0, NJ1)
                cb = jnp.where(c == 0, s, NS * NJ0 + s)
                pltpu.sync_copy(ei_hbm.at[cb], idxr.at[0])
                pltpu.async_copy(g.at[idxr.at[0, 0]], rows.at[0], sem_g)
                pltpu.async_copy(ei_hbm.at[cb + NS], idxr.at[1], sem_i)

                def body_j(j, carry, g=g):
                    b = lax.rem(j, 2)
                    nb = 1 - b
                    sl1 = lax.rem(j + 1, 3)
                    sl2 = lax.rem(j + 2, 3)

                    @pl.when(j + 2 < njc)
                    def _():
                        pltpu.async_copy(
                            ei_hbm.at[cb + NS * (j + 2)], idxr.at[sl2], sem_i)

                    pltpu.make_async_copy(
                        ei_hbm.at[cb + NS * (j + 1)], idxr.at[sl1], sem_i).wait()
                    pltpu.async_copy(g.at[idxr.at[sl1, 0]], rows.at[nb], sem_g)
                    pltpu.make_async_copy(
                        g.at[idxr.at[sl1, 0]], rows.at[b], sem_g).wait()
                    pltpu.sync_copy(
                        rows.at[b], acc.at[idxr.at[lax.rem(j, 3), 1]], add=True)
                    return carry

                lax.fori_loop(0, njc - 1, body_j, 0)
                last = njc - 1
                lb = lax.rem(last, 2)
                slL = lax.rem(last, 3)
                pltpu.make_async_copy(
                    g.at[idxr.at[slL, 0]], rows.at[lb], sem_g).wait()
                pltpu.sync_copy(
                    rows.at[lb], acc.at[idxr.at[slL, 1]], add=True)

            _sweep()
            plsc.subcore_barrier()
            pltpu.sync_copy(acc.at[pl.ds(s * DPT, DPT)],
                            out_hbm.at[t, c, pl.ds(s * DPT, DPT)])
            plsc.subcore_barrier()

    return pl.kernel(
        body,
        out_type=jax.ShapeDtypeStruct((ntab, NC, NPAD, D), jnp.float32),
        mesh=_MESH,
        scratch_types=[
            pltpu.VMEM((3, 2, CHUNK), jnp.int32),
            pltpu.VMEM((2, CHUNK, D), jnp.float32),
            pltpu.VMEM((64, D), jnp.float32),
            pltpu.SemaphoreType.DMA,
            pltpu.SemaphoreType.DMA,
            pltpu.SemaphoreType.DMA,
            pltpu.VMEM_SHARED((NPAD, D), jnp.float32),
        ],
    )


_agg1 = _make_agg(1)
_agg4 = _make_agg(4)

_DOT = dict(preferred_element_type=jnp.float32, precision=lax.Precision.HIGHEST)


def _tc1_body(deg_ref, x_ref, w_ref, b_ref, inv_ref, g1_ref):
    dsum = deg_ref[0] + deg_ref[1] + 1.0
    iv = lax.rsqrt(dsum)
    inv_ref[...] = iv
    hl = jnp.dot(x_ref[...], w_ref[...], **_DOT) + b_ref[...]
    g1_ref[...] = hl * iv


_tc1 = pl.pallas_call(
    _tc1_body,
    grid=(GRID,),
    in_specs=[
        pl.BlockSpec((NC, BN, 1), lambda i: (0, i, 0)),
        pl.BlockSpec((BN, D), lambda i: (i, 0)),
        pl.BlockSpec((D, D), lambda i: (0, 0)),
        pl.BlockSpec((1, D), lambda i: (0, 0)),
    ],
    out_specs=[
        pl.BlockSpec((BN, 1), lambda i: (i, 0)),
        pl.BlockSpec((BN, D), lambda i: (i, 0)),
    ],
    out_shape=[
        jax.ShapeDtypeStruct((N, 1), jnp.float32),
        jax.ShapeDtypeStruct((N, D), jnp.float32),
    ],
)


def _tc2_body(p_ref, g1_ref, inv_ref, w_ref, b_ref, g2_ref, fl_ref):
    i = pl.program_id(0)

    @pl.when(i == 0)
    def _():
        fl_ref[...] = jnp.zeros_like(fl_ref)

    iv = inv_ref[...]
    cur = iv * (p_ref[0] + p_ref[1] + g1_ref[...])
    v = jnp.zeros_like(cur)
    for t in range(4):
        v = v + (cur - v) * 0.5
        sp = (v >= 1.0).astype(jnp.float32)
        v = v - sp
        g2t = (jnp.dot(sp, w_ref[...], **_DOT) + b_ref[...]) * iv
        g2_ref[t] = g2t
        fl_ref[t] = jnp.maximum(fl_ref[t], jnp.max(jnp.abs(g2t)))


_tc2 = pl.pallas_call(
    _tc2_body,
    grid=(GRID,),
    in_specs=[
        pl.BlockSpec((NC, BN, D), lambda i: (0, i, 0)),
        pl.BlockSpec((BN, D), lambda i: (i, 0)),
        pl.BlockSpec((BN, 1), lambda i: (i, 0)),
        pl.BlockSpec((D, D), lambda i: (0, 0)),
        pl.BlockSpec((1, D), lambda i: (0, 0)),
    ],
    out_specs=[
        pl.BlockSpec((4, BN, D), lambda i: (0, i, 0)),
        pl.BlockSpec((4, 8, CHUNK), lambda i: (0, 0, 0)),
    ],
    out_shape=[
        jax.ShapeDtypeStruct((4, N, D), jnp.float32),
        jax.ShapeDtypeStruct((4, 8, CHUNK), jnp.float32),
    ],
)


def _make_tc3(with_pp):
    def body(*refs):
        if with_pp:
            pp_ref, g2_ref, inv_ref, w_ref, b_ref, out_ref = refs
        else:
            g2_ref, inv_ref, w_ref, b_ref, out_ref = refs
        iv = inv_ref[...]
        v = jnp.zeros((BN, D), jnp.float32)
        acc = jnp.zeros((BN, D), jnp.float32)
        for t in range(4):
            if with_pp:
                cur = iv * (pp_ref[t, 0] + pp_ref[t, 1] + g2_ref[t])
            else:
                cur = iv * g2_ref[t]
            v = v + (cur - v) * 0.5
            sp = (v >= 1.0).astype(jnp.float32)
            v = v - sp
            acc = acc + sp
        out_ref[...] = jnp.dot(acc * 0.25, w_ref[...], **_DOT) + b_ref[...]

    in_specs = [
        pl.BlockSpec((4, BN, D), lambda i: (0, i, 0)),
        pl.BlockSpec((BN, 1), lambda i: (i, 0)),
        pl.BlockSpec((D, D), lambda i: (0, 0)),
        pl.BlockSpec((1, D), lambda i: (0, 0)),
    ]
    if with_pp:
        in_specs = [pl.BlockSpec((4, NC, BN, D), lambda i: (0, 0, i, 0))] + in_specs
    return pl.pallas_call(
        body,
        grid=(GRID,),
        in_specs=in_specs,
        out_specs=pl.BlockSpec((BN, D), lambda i: (i, 0)),
        out_shape=jax.ShapeDtypeStruct((N, D), jnp.float32),
    )


_tc3p = _make_tc3(True)
_tc3z = _make_tc3(False)


def kernel(x, edge_index, W0, b0, W1, b1, W_out, b_out):
    srcp = jnp.concatenate(
        [edge_index[0], jnp.zeros((EPAD,), jnp.int32)]).reshape(NCHUNKP, CHUNK)
    pad_dst = N + jax.lax.rem(jnp.arange(EPAD, dtype=jnp.int32), jnp.int32(128))
    dstp = jnp.concatenate(
        [edge_index[1], pad_dst]).reshape(NCHUNKP, CHUNK)
    ei = jnp.stack([srcp, dstp], axis=1)
    degp = _deg_call(ei).reshape(NC, NPAD, 1)
    inv, g1 = _tc1(degp, x, W0, b0.reshape(1, D))
    p1 = _agg1(g1, ei).reshape(NC, NPAD, D)
    g2, fl = _tc2(p1, g1, inv, W1, b1.reshape(1, D))
    bo = b_out.reshape(1, D)

    def _with_spikes():
        pp = _agg4(g2[0], g2[1], g2[2], g2[3], ei)
        return _tc3p(pp, g2, inv, W_out, bo)

    def _no_spikes():
        return _tc3z(g2, inv, W_out, bo)

    return lax.cond(fl[:, 0, 0].max() > 0.0, _with_spikes, _no_spikes)
